# baseline (speedup 1.0000x reference)
import functools

import jax
import jax.numpy as jnp
from jax import lax
from jax.experimental import pallas as pl
from jax.experimental.pallas import tpu as pltpu
from jax.experimental.pallas import tpu_sc as plsc

_NC = 2
_NS = 16
_L = 16


def _k1_body(q_ref, k_ref, w_ref, gi_ref, *, S, NB):
    q = q_ref[0]
    k = k_ref[0]
    dot = lax.dot_general(k, q, (((1,), (1,)), ((), ())),
                          preferred_element_type=jnp.float32)
    kn = jnp.sum(k * k, axis=1, keepdims=True)
    ones8 = jnp.ones((8, q.shape[1]), jnp.float32)
    qn8 = lax.dot_general(ones8, q * q, (((1,), (1,)), ((), ())),
                          preferred_element_type=jnp.float32)
    dist = kn + qn8[0:1, :] - 2.0 * dot

    inf = jnp.float32(jnp.inf)
    m0 = jnp.min(dist, axis=0, keepdims=True)
    c0 = dist == m0
    d1 = jnp.where(c0, inf, dist)
    m1 = jnp.min(d1, axis=0, keepdims=True)
    c1 = d1 == m1
    d2 = jnp.where(c1, inf, d1)
    m2 = jnp.min(d2, axis=0, keepdims=True)
    c2 = d2 == m2

    ii = lax.broadcasted_iota(jnp.int32, (S, NB), 0)
    i0 = jnp.min(jnp.where(c0, ii, S), axis=0, keepdims=True)
    i1 = jnp.min(jnp.where(c1, ii, S), axis=0, keepdims=True)
    i2 = jnp.min(jnp.where(c2, ii, S), axis=0, keepdims=True)

    r0 = 1.0 / (m0 + 1e-8)
    r1 = 1.0 / (m1 + 1e-8)
    r2 = 1.0 / (m2 + 1e-8)
    rnorm = r0 + r1 + r2

    zrow = jnp.zeros((5, NB), jnp.float32)
    w8 = jnp.concatenate([r0 / rnorm, r1 / rnorm, r2 / rnorm, zrow],
                         axis=0)
    eye = jnp.eye(8, 16, dtype=jnp.float32)
    w_ref[0] = lax.dot_general(w8, eye, (((0,), (0,)), ((), ())),
                               preferred_element_type=jnp.float32)
    goff = pl.program_id(0) * S
    zrow_i = jnp.zeros((5, NB), jnp.int32)
    gi_ref[0] = jnp.concatenate([i0 + goff, i1 + goff, i2 + goff, zrow_i],
                                axis=0)


def _sc_interp_body(table_ref, gi_ref, w_ref, out_ref,
                    i0a, i1a, i2a, w_a, r0a, r1a, r2a,
                    i0b, i1b, i2b, w_b, r0b, r1b, r2b,
                    out_v, sema, semb,
                    *, N, D2, RPW, C):
    wid = lax.axis_index("s") * _NC + lax.axis_index("c")
    wpb = N // RPW
    bh = wid // wpb
    n0 = (wid % wpb) * RPW
    nchunks = RPW // C
    nj = D2 // _L
    bufs = ((i0a, i1a, i2a, w_a, r0a, r1a, r2a, sema),
            (i0b, i1b, i2b, w_b, r0b, r1b, r2b, semb))

    def issue(c, p):
        i0, i1, i2, w_v, r0, r1, r2, sem = bufs[p]
        nbase = n0 + c * C
        pltpu.sync_copy(gi_ref.at[bh * 8 + 0, pl.ds(nbase, C)], i0)
        pltpu.sync_copy(gi_ref.at[bh * 8 + 1, pl.ds(nbase, C)], i1)
        pltpu.sync_copy(gi_ref.at[bh * 8 + 2, pl.ds(nbase, C)], i2)
        pltpu.sync_copy(w_ref.at[pl.ds(bh * N + nbase, C)], w_v)
        return (pltpu.async_copy(table_ref.at[i0], r0, sem),
                pltpu.async_copy(table_ref.at[i1], r1, sem),
                pltpu.async_copy(table_ref.at[i2], r2, sem))

    def consume(c, p, cps):
        i0, i1, i2, w_v, r0, r1, r2, sem = bufs[p]
        for cp in cps:
            cp.wait()

        def row_body(n, _):
            wrow = w_v[n, pl.ds(0, _L)]
            w0n = jnp.full((_L,), wrow[0], jnp.float32)
            w1n = jnp.full((_L,), wrow[1], jnp.float32)
            w2n = jnp.full((_L,), wrow[2], jnp.float32)
            for j in range(nj):
                sl = pl.ds(j * _L, _L)
                out_v[n, sl] = (w0n * r0[n, sl] + w1n * r1[n, sl]
                                + w2n * r2[n, sl])
            return 0

        lax.fori_loop(0, C, row_body, 0)
        pltpu.sync_copy(out_v, out_ref.at[pl.ds(bh * N + n0 + c * C, C)])

    def pair_body(g, _):
        c0 = 2 * g
        cpa = issue(c0, 0)
        cpb = issue(c0 + 1, 1)
        consume(c0, 0, cpa)
        consume(c0 + 1, 1, cpb)
        return 0

    lax.fori_loop(0, nchunks // 2, pair_body, 0)


def _k2_body(ip_ref, p1_ref, w0p_ref, w0i_ref, z1_ref, sums_ref):
    ip = ip_ref[0]
    z1 = (lax.dot_general(w0i_ref[...], ip, (((1,), (1,)), ((), ())),
                          preferred_element_type=jnp.float32)
          + lax.dot_general(w0p_ref[...], p1_ref[0], (((1,), (0,)), ((), ())),
                            preferred_element_type=jnp.float32))
    z1_ref[0] = z1
    part = jnp.concatenate(
        [jnp.sum(z1, axis=1, keepdims=True),
         jnp.sum(z1 * z1, axis=1, keepdims=True)], axis=1)
    first = (pl.program_id(0) == 0) & (pl.program_id(1) == 0)

    @pl.when(first)
    def _():
        sums_ref[...] = part

    @pl.when(jnp.logical_not(first))
    def _():
        sums_ref[...] = sums_ref[...] + part


def _k3_body(z1_ref, sums_ref, g_ref, be_ref, w1_ref, z2_ref, sums2_ref,
             *, count):
    s = sums_ref[:, 0:1]
    sq = sums_ref[:, 1:2]
    mu = s / count
    var = sq / count - mu * mu
    inv = g_ref[...] * lax.rsqrt(var + 1e-5)
    shift = be_ref[...] - mu * inv
    z1 = z1_ref[0]
    h = jnp.maximum(z1 * inv + shift, 0.0)
    z2 = lax.dot_general(w1_ref[...], h, (((1,), (0,)), ((), ())),
                         preferred_element_type=jnp.float32)
    z2_ref[0] = z2
    part = jnp.concatenate(
        [jnp.sum(z2, axis=1, keepdims=True),
         jnp.sum(z2 * z2, axis=1, keepdims=True)], axis=1)
    first = (pl.program_id(0) == 0) & (pl.program_id(1) == 0)

    @pl.when(first)
    def _():
        sums2_ref[...] = part

    @pl.when(jnp.logical_not(first))
    def _():
        sums2_ref[...] = sums2_ref[...] + part


def _k4_body(z2_ref, sums_ref, g_ref, be_ref, y_ref, *, count):
    s = sums_ref[:, 0:1]
    sq = sums_ref[:, 1:2]
    mu = s / count
    var = sq / count - mu * mu
    inv = g_ref[...] * lax.rsqrt(var + 1e-5)
    shift = be_ref[...] - mu * inv
    z2 = z2_ref[0]
    y_ref[0] = jnp.maximum(z2 * inv + shift, 0.0)


def kernel(xyz1, xyz2, points1, points2, feat1, feat2,
           W0, b0, g0, be0, W1, b1, g1, be1):
    B, _, N = xyz1.shape
    S = xyz2.shape[2]
    D1 = points1.shape[1]
    D2 = points2.shape[1]
    CF = feat1.shape[2]
    C1 = W0.shape[0]
    C2 = W1.shape[0]
    NB = min(256, N)
    NJ = N // NB
    C = 3 + CF
    CP = ((C + 7) // 8) * 8

    x1t = jnp.transpose(xyz1, (0, 2, 1))
    x2t = jnp.transpose(xyz2, (0, 2, 1))
    q = jnp.concatenate(
        [x1t, feat1, jnp.zeros((B, N, CP - C), jnp.float32)], axis=2)
    kq = jnp.concatenate(
        [x2t, feat2, jnp.zeros((B, S, CP - C), jnp.float32)], axis=2)
    table = jnp.transpose(points2, (0, 2, 1)).reshape(B * S, D2)
    p1b = points1
    w0p = W0[:, :D1]
    w0i = W0[:, D1:]
    w1b = W1
    g0c = g0[:, None]
    be0c = be0[:, None]
    g1c = g1[:, None]
    be1c = be1[:, None]

    wh, gih = pl.pallas_call(
        functools.partial(_k1_body, S=S, NB=NB),
        grid=(B, NJ),
        in_specs=[
            pl.BlockSpec((1, NB, CP), lambda b, j: (b, j, 0)),
            pl.BlockSpec((1, S, CP), lambda b, j: (b, 0, 0)),
        ],
        out_specs=[
            pl.BlockSpec((1, NB, 16), lambda b, j: (b, j, 0)),
            pl.BlockSpec((1, 8, NB), lambda b, j: (b, 0, j)),
        ],
        out_shape=[
            jax.ShapeDtypeStruct((B, N, 16), jnp.float32),
            jax.ShapeDtypeStruct((B, 8, N), jnp.int32),
        ],
    )(q, kq)

    R = B * N
    NW = _NC * _NS
    RPW = R // NW
    CC = 32
    sc = functools.partial(
        pl.kernel,
        mesh=plsc.VectorSubcoreMesh(core_axis_name="c",
                                    subcore_axis_name="s"),
        out_type=jax.ShapeDtypeStruct((R, D2), jnp.float32),
        scratch_types=(
            2 * [pltpu.VMEM((CC,), jnp.int32),
                 pltpu.VMEM((CC,), jnp.int32),
                 pltpu.VMEM((CC,), jnp.int32),
                 pltpu.VMEM((CC, 16), jnp.float32),
                 pltpu.VMEM((CC, D2), jnp.float32),
                 pltpu.VMEM((CC, D2), jnp.float32),
                 pltpu.VMEM((CC, D2), jnp.float32)]
            + [pltpu.VMEM((CC, D2), jnp.float32),
               pltpu.SemaphoreType.DMA,
               pltpu.SemaphoreType.DMA]
        ),
    )(functools.partial(_sc_interp_body, N=N, D2=D2, RPW=RPW, C=CC))
    interp = sc(table, gih.reshape(B * 8, N),
                wh.reshape(B * N, 16)).reshape(B, N, D2)

    z1, sums1 = pl.pallas_call(
        _k2_body,
        grid=(B, NJ),
        in_specs=[
            pl.BlockSpec((1, NB, D2), lambda b, j: (b, j, 0)),
            pl.BlockSpec((1, D1, NB), lambda b, j: (b, 0, j)),
            pl.BlockSpec((C1, D1), lambda b, j: (0, 0)),
            pl.BlockSpec((C1, D2), lambda b, j: (0, 0)),
        ],
        out_specs=[
            pl.BlockSpec((1, C1, NB), lambda b, j: (b, 0, j)),
            pl.BlockSpec((C1, 2), lambda b, j: (0, 0)),
        ],
        out_shape=[
            jax.ShapeDtypeStruct((B, C1, N), jnp.float32),
            jax.ShapeDtypeStruct((C1, 2), jnp.float32),
        ],
    )(interp, p1b, w0p, w0i)

    count = float(B * N)
    z2, sums2 = pl.pallas_call(
        functools.partial(_k3_body, count=count),
        grid=(B, NJ),
        in_specs=[
            pl.BlockSpec((1, C1, NB), lambda b, j: (b, 0, j)),
            pl.BlockSpec((C1, 2), lambda b, j: (0, 0)),
            pl.BlockSpec((C1, 1), lambda b, j: (0, 0)),
            pl.BlockSpec((C1, 1), lambda b, j: (0, 0)),
            pl.BlockSpec((C2, C1), lambda b, j: (0, 0)),
        ],
        out_specs=[
            pl.BlockSpec((1, C2, NB), lambda b, j: (b, 0, j)),
            pl.BlockSpec((C2, 2), lambda b, j: (0, 0)),
        ],
        out_shape=[
            jax.ShapeDtypeStruct((B, C2, N), jnp.float32),
            jax.ShapeDtypeStruct((C2, 2), jnp.float32),
        ],
    )(z1, sums1, g0c, be0c, w1b)

    y = pl.pallas_call(
        functools.partial(_k4_body, count=count),
        grid=(B, NJ),
        in_specs=[
            pl.BlockSpec((1, C2, NB), lambda b, j: (b, 0, j)),
            pl.BlockSpec((C2, 2), lambda b, j: (0, 0)),
            pl.BlockSpec((C2, 1), lambda b, j: (0, 0)),
            pl.BlockSpec((C2, 1), lambda b, j: (0, 0)),
        ],
        out_specs=pl.BlockSpec((1, C2, NB), lambda b, j: (b, 0, j)),
        out_shape=jax.ShapeDtypeStruct((B, C2, N), jnp.float32),
    )(z2, sums2, g1c, be1c)

    return y


# speedup vs baseline: 1.2006x; 1.2006x over previous
import functools

import jax
import jax.numpy as jnp
from jax import lax
from jax.experimental import pallas as pl
from jax.experimental.pallas import tpu as pltpu
from jax.experimental.pallas import tpu_sc as plsc

_NC = 2
_NS = 16
_L = 16


def _k1_body(q_ref, k_ref, w_ref, gi_ref, *, S, NB, h, B2):
    q = q_ref[0]
    k = k_ref[0]
    dot = lax.dot_general(k, q, (((1,), (1,)), ((), ())),
                          preferred_element_type=jnp.float32)
    kn = jnp.sum(k * k, axis=1, keepdims=True)
    ones8 = jnp.ones((8, q.shape[1]), jnp.float32)
    qn8 = lax.dot_general(ones8, q * q, (((1,), (1,)), ((), ())),
                          preferred_element_type=jnp.float32)
    dist = kn + qn8[0:1, :] - 2.0 * dot

    inf = jnp.float32(jnp.inf)
    m0 = jnp.min(dist, axis=0, keepdims=True)
    c0 = dist == m0
    d1 = jnp.where(c0, inf, dist)
    m1 = jnp.min(d1, axis=0, keepdims=True)
    c1 = d1 == m1
    d2 = jnp.where(c1, inf, d1)
    m2 = jnp.min(d2, axis=0, keepdims=True)
    c2 = d2 == m2

    ii = lax.broadcasted_iota(jnp.int32, (S, NB), 0)
    i0 = jnp.min(jnp.where(c0, ii, S), axis=0, keepdims=True)
    i1 = jnp.min(jnp.where(c1, ii, S), axis=0, keepdims=True)
    i2 = jnp.min(jnp.where(c2, ii, S), axis=0, keepdims=True)

    r0 = 1.0 / (m0 + 1e-8)
    r1 = 1.0 / (m1 + 1e-8)
    r2 = 1.0 / (m2 + 1e-8)
    rnorm = r0 + r1 + r2

    zrow = jnp.zeros((5, NB), jnp.float32)
    w8 = jnp.concatenate([r0 / rnorm, r1 / rnorm, r2 / rnorm, zrow],
                         axis=0)
    eye = jnp.eye(8, 16, dtype=jnp.float32)
    w_ref[0] = lax.dot_general(w8, eye, (((0,), (0,)), ((), ())),
                               preferred_element_type=jnp.float32)
    goff = (pl.program_id(0) + h * B2) * S
    zrow_i = jnp.zeros((5, NB), jnp.int32)
    gi_ref[0] = jnp.concatenate([i0 + goff, i1 + goff, i2 + goff, zrow_i],
                                axis=0)


def _sc_interp_body(table_ref, gi_ref, w_ref, out_ref,
                    i0a, i1a, i2a, w_a, r0a, r1a, r2a,
                    i0b, i1b, i2b, w_b, r0b, r1b, r2b,
                    out_v, sema, semb,
                    *, N, D2, RPW, C):
    wid = lax.axis_index("s") * _NC + lax.axis_index("c")
    wpb = N // RPW
    bh = wid // wpb
    n0 = (wid % wpb) * RPW
    nchunks = RPW // C
    nj = D2 // _L
    bufs = ((i0a, i1a, i2a, w_a, r0a, r1a, r2a, sema),
            (i0b, i1b, i2b, w_b, r0b, r1b, r2b, semb))

    def issue(c, p):
        i0, i1, i2, w_v, r0, r1, r2, sem = bufs[p]
        nbase = n0 + c * C
        pltpu.sync_copy(gi_ref.at[bh * 8 + 0, pl.ds(nbase, C)], i0)
        pltpu.sync_copy(gi_ref.at[bh * 8 + 1, pl.ds(nbase, C)], i1)
        pltpu.sync_copy(gi_ref.at[bh * 8 + 2, pl.ds(nbase, C)], i2)
        pltpu.sync_copy(w_ref.at[pl.ds(bh * N + nbase, C)], w_v)
        return (pltpu.async_copy(table_ref.at[i0], r0, sem),
                pltpu.async_copy(table_ref.at[i1], r1, sem),
                pltpu.async_copy(table_ref.at[i2], r2, sem))

    def consume(c, p, cps):
        i0, i1, i2, w_v, r0, r1, r2, sem = bufs[p]
        for cp in cps:
            cp.wait()

        def row_body(n, _):
            wrow = w_v[n, pl.ds(0, _L)]
            w0n = jnp.full((_L,), wrow[0], jnp.float32)
            w1n = jnp.full((_L,), wrow[1], jnp.float32)
            w2n = jnp.full((_L,), wrow[2], jnp.float32)
            for j in range(nj):
                sl = pl.ds(j * _L, _L)
                out_v[n, sl] = (w0n * r0[n, sl] + w1n * r1[n, sl]
                                + w2n * r2[n, sl])
            return 0

        lax.fori_loop(0, C, row_body, 0)
        pltpu.sync_copy(out_v, out_ref.at[pl.ds(bh * N + n0 + c * C, C)])

    def pair_body(g, _):
        c0 = 2 * g
        cpa = issue(c0, 0)
        cpb = issue(c0 + 1, 1)
        consume(c0, 0, cpa)
        consume(c0 + 1, 1, cpb)
        return 0

    lax.fori_loop(0, nchunks // 2, pair_body, 0)


def _k2_body(ip_ref, p1_ref, w0p_ref, w0i_ref, z1_ref, sums_ref):
    ip = ip_ref[0].astype(jnp.bfloat16)
    z1 = (lax.dot_general(w0i_ref[...], ip, (((1,), (1,)), ((), ())),
                          preferred_element_type=jnp.float32)
          + lax.dot_general(w0p_ref[...], p1_ref[0], (((1,), (0,)), ((), ())),
                            preferred_element_type=jnp.float32))
    z1_ref[0] = z1.astype(jnp.bfloat16)
    part = jnp.concatenate(
        [jnp.sum(z1, axis=1, keepdims=True),
         jnp.sum(z1 * z1, axis=1, keepdims=True)], axis=1)
    first = (pl.program_id(0) == 0) & (pl.program_id(1) == 0)

    @pl.when(first)
    def _():
        sums_ref[...] = part

    @pl.when(jnp.logical_not(first))
    def _():
        sums_ref[...] = sums_ref[...] + part


def _k3_body(z1a_ref, z1b_ref, sums_ref, g_ref, be_ref, w1_ref,
             z2_ref, sums2_ref, *, count, B2):
    s = sums_ref[:, 0:1]
    sq = sums_ref[:, 1:2]
    mu = s / count
    var = sq / count - mu * mu
    inv = g_ref[...] * lax.rsqrt(var + 1e-5)
    shift = be_ref[...] - mu * inv
    sel = pl.program_id(0) < B2
    z1 = jnp.where(sel, z1a_ref[0], z1b_ref[0]).astype(jnp.float32)
    h = jnp.maximum(z1 * inv + shift, 0.0).astype(jnp.bfloat16)
    z2 = lax.dot_general(w1_ref[...], h, (((1,), (0,)), ((), ())),
                         preferred_element_type=jnp.float32)
    z2_ref[0] = z2.astype(jnp.bfloat16)
    part = jnp.concatenate(
        [jnp.sum(z2, axis=1, keepdims=True),
         jnp.sum(z2 * z2, axis=1, keepdims=True)], axis=1)
    first = (pl.program_id(0) == 0) & (pl.program_id(1) == 0)

    @pl.when(first)
    def _():
        sums2_ref[...] = part

    @pl.when(jnp.logical_not(first))
    def _():
        sums2_ref[...] = sums2_ref[...] + part


def _k4_body(z2_ref, sums_ref, g_ref, be_ref, y_ref, *, count):
    s = sums_ref[:, 0:1]
    sq = sums_ref[:, 1:2]
    mu = s / count
    var = sq / count - mu * mu
    inv = g_ref[...] * lax.rsqrt(var + 1e-5)
    shift = be_ref[...] - mu * inv
    z2 = z2_ref[0].astype(jnp.float32)
    y_ref[0] = jnp.maximum(z2 * inv + shift, 0.0)


def kernel(xyz1, xyz2, points1, points2, feat1, feat2,
           W0, b0, g0, be0, W1, b1, g1, be1):
    B, _, N = xyz1.shape
    S = xyz2.shape[2]
    D1 = points1.shape[1]
    D2 = points2.shape[1]
    CF = feat1.shape[2]
    C1 = W0.shape[0]
    C2 = W1.shape[0]
    NB = min(256, N)
    NJ = N // NB
    C = 3 + CF
    CP = ((C + 7) // 8) * 8

    x1t = jnp.transpose(xyz1, (0, 2, 1))
    x2t = jnp.transpose(xyz2, (0, 2, 1))
    q = jnp.concatenate(
        [x1t, feat1, jnp.zeros((B, N, CP - C), jnp.float32)], axis=2)
    kq = jnp.concatenate(
        [x2t, feat2, jnp.zeros((B, S, CP - C), jnp.float32)], axis=2)
    table = jnp.transpose(points2, (0, 2, 1)).reshape(B * S, D2)
    p1b = points1.astype(jnp.bfloat16)
    w0p = W0[:, :D1].astype(jnp.bfloat16)
    w0i = W0[:, D1:].astype(jnp.bfloat16)
    w1b = W1.astype(jnp.bfloat16)
    g0c = g0[:, None]
    be0c = be0[:, None]
    g1c = g1[:, None]
    be1c = be1[:, None]

    B2 = B // 2
    R = B2 * N
    NW = _NC * _NS
    RPW = R // NW
    CC = 32
    z1s = []
    sums1 = None
    for h in range(2):
        wh, gih = pl.pallas_call(
            functools.partial(_k1_body, S=S, NB=NB, h=h, B2=B2),
            grid=(B2, NJ),
            in_specs=[
                pl.BlockSpec((1, NB, CP),
                             lambda b, j, h=h: (b + h * B2, j, 0)),
                pl.BlockSpec((1, S, CP),
                             lambda b, j, h=h: (b + h * B2, 0, 0)),
            ],
            out_specs=[
                pl.BlockSpec((1, NB, 16), lambda b, j: (b, j, 0)),
                pl.BlockSpec((1, 8, NB), lambda b, j: (b, 0, j)),
            ],
            out_shape=[
                jax.ShapeDtypeStruct((B2, N, 16), jnp.float32),
                jax.ShapeDtypeStruct((B2, 8, N), jnp.int32),
            ],
        )(q, kq)

        sc = functools.partial(
            pl.kernel,
            mesh=plsc.VectorSubcoreMesh(core_axis_name="c",
                                        subcore_axis_name="s"),
            out_type=jax.ShapeDtypeStruct((R, D2), jnp.float32),
            scratch_types=(
                2 * [pltpu.VMEM((CC,), jnp.int32),
                     pltpu.VMEM((CC,), jnp.int32),
                     pltpu.VMEM((CC,), jnp.int32),
                     pltpu.VMEM((CC, 16), jnp.float32),
                     pltpu.VMEM((CC, D2), jnp.float32),
                     pltpu.VMEM((CC, D2), jnp.float32),
                     pltpu.VMEM((CC, D2), jnp.float32)]
                + [pltpu.VMEM((CC, D2), jnp.float32),
                   pltpu.SemaphoreType.DMA,
                   pltpu.SemaphoreType.DMA]
            ),
        )(functools.partial(_sc_interp_body, N=N, D2=D2, RPW=RPW, C=CC))
        interp = sc(table, gih.reshape(B2 * 8, N),
                    wh.reshape(B2 * N, 16)).reshape(B2, N, D2)

        z1h, s1h = pl.pallas_call(
            _k2_body,
            grid=(B2, NJ),
            in_specs=[
                pl.BlockSpec((1, NB, D2), lambda b, j: (b, j, 0)),
                pl.BlockSpec((1, D1, NB),
                             lambda b, j, h=h: (b + h * B2, 0, j)),
                pl.BlockSpec((C1, D1), lambda b, j: (0, 0)),
                pl.BlockSpec((C1, D2), lambda b, j: (0, 0)),
            ],
            out_specs=[
                pl.BlockSpec((1, C1, NB), lambda b, j: (b, 0, j)),
                pl.BlockSpec((C1, 2), lambda b, j: (0, 0)),
            ],
            out_shape=[
                jax.ShapeDtypeStruct((B2, C1, N), jnp.bfloat16),
                jax.ShapeDtypeStruct((C1, 2), jnp.float32),
            ],
        )(interp, p1b, w0p, w0i)
        z1s.append(z1h)
        sums1 = s1h if sums1 is None else sums1 + s1h

    count = float(B * N)
    z2, sums2 = pl.pallas_call(
        functools.partial(_k3_body, count=count, B2=B2),
        grid=(B, NJ),
        in_specs=[
            pl.BlockSpec((1, C1, NB), lambda b, j: (b % B2, 0, j)),
            pl.BlockSpec((1, C1, NB), lambda b, j: (b % B2, 0, j)),
            pl.BlockSpec((C1, 2), lambda b, j: (0, 0)),
            pl.BlockSpec((C1, 1), lambda b, j: (0, 0)),
            pl.BlockSpec((C1, 1), lambda b, j: (0, 0)),
            pl.BlockSpec((C2, C1), lambda b, j: (0, 0)),
        ],
        out_specs=[
            pl.BlockSpec((1, C2, NB), lambda b, j: (b, 0, j)),
            pl.BlockSpec((C2, 2), lambda b, j: (0, 0)),
        ],
        out_shape=[
            jax.ShapeDtypeStruct((B, C2, N), jnp.bfloat16),
            jax.ShapeDtypeStruct((C2, 2), jnp.float32),
        ],
    )(z1s[0], z1s[1], sums1, g0c, be0c, w1b)

    y = pl.pallas_call(
        functools.partial(_k4_body, count=count),
        grid=(B, NJ),
        in_specs=[
            pl.BlockSpec((1, C2, NB), lambda b, j: (b, 0, j)),
            pl.BlockSpec((C2, 2), lambda b, j: (0, 0)),
            pl.BlockSpec((C2, 1), lambda b, j: (0, 0)),
            pl.BlockSpec((C2, 1), lambda b, j: (0, 0)),
        ],
        out_specs=pl.BlockSpec((1, C2, NB), lambda b, j: (b, 0, j)),
        out_shape=jax.ShapeDtypeStruct((B, C2, N), jnp.float32),
    )(z2, sums2, g1c, be1c)

    return y
